# baseline (speedup 1.0000x reference)
import functools

import jax
import jax.numpy as jnp
from jax import lax
from jax.experimental import pallas as pl
from jax.experimental.pallas import tpu as pltpu
from jax.experimental.pallas import tpu_sc as plsc

B, C_IN, C_OUT, H, W, K = 4, 192, 384, 28, 28, 9
N = H * W
BN = B * N
KN = K - 1

NW = 32
HC = C_OUT // 2
HCP = 256
CR = 16
NCH = BN // CR
IPC = CR * K
IPH = IPC // 2


def _key16(v):
    bits = lax.bitcast_convert_type(v, jnp.int32)
    rnd = jnp.bitwise_and(lax.shift_right_logical(bits, jnp.int32(16)),
                          jnp.int32(1)) + jnp.int32(0x7FFF)
    u = lax.shift_right_logical(bits + rnd, jnp.int32(16))
    s = lax.shift_right_logical(u, jnp.int32(15))
    flip = jnp.bitwise_or(jnp.bitwise_and(jnp.int32(0) - s, jnp.int32(0xFFFF)),
                          jnp.int32(0x8000))
    return jnp.bitwise_xor(u, flip)


def _unkey16(k):
    u = jnp.where(k >= jnp.int32(0x8000),
                  jnp.bitwise_xor(k, jnp.int32(0x8000)),
                  jnp.bitwise_xor(k, jnp.int32(0xFFFF)))
    return lax.bitcast_convert_type(lax.shift_left(u, jnp.int32(16)),
                                    jnp.float32)


def _t1_body(x_ref, wt_ref, b_ref, idx_ref, a_ref, yt_ref):
    bidx = pl.program_id(0)
    x = x_ref[0]
    s = jnp.sum(x * x, axis=0, keepdims=True)
    xn = x / (jnp.sqrt(s) + 1e-12)

    g = lax.dot_general(xn, xn, (((0,), (0,)), ((), ())),
                        preferred_element_type=jnp.float32)
    sq = jnp.sum(xn * xn, axis=0, keepdims=True)
    dist = jnp.transpose(sq) + (-2.0 * g) + sq

    rows = lax.broadcasted_iota(jnp.int32, (N, N), 0)
    cols = lax.broadcasted_iota(jnp.int32, (N, N), 1)

    dq = lax.convert_element_type(
        jnp.minimum(jnp.maximum(dist, 0.0), 3.9) * 524288.0, jnp.int32)
    key_i = jnp.bitwise_or(lax.shift_left(dq, jnp.int32(10)), cols)
    bigf = lax.bitcast_convert_type(jnp.int32(0x7F000000), jnp.float32)
    key = jnp.where(rows == cols, bigf,
                    lax.bitcast_convert_type(key_i, jnp.float32))

    picked = []
    for _ in range(KN):
        m = jnp.min(key, axis=1, keepdims=True)
        am = jnp.bitwise_and(lax.bitcast_convert_type(m, jnp.int32),
                             jnp.int32(0x3FF))
        picked.append(am + bidx * N)
        key = jnp.where(key == m, bigf, key)
    picked.append(lax.broadcasted_iota(jnp.int32, (N, 1), 0) + bidx * N)
    idx_ref[0] = jnp.concatenate(picked, axis=1)

    w1 = wt_ref[:, :C_IN]
    w2 = wt_ref[:, C_IN:]
    a = lax.dot_general(w1 - w2, x, (((1,), (0,)), ((), ())),
                        preferred_element_type=jnp.float32)
    a_ref[0] = (a + b_ref[...]).astype(jnp.bfloat16)
    ya = lax.dot_general(x, w2[:HC], (((0,), (1,)), ((), ())),
                         preferred_element_type=jnp.float32)
    yb = lax.dot_general(x, w2[HC:], (((0,), (1,)), ((), ())),
                         preferred_element_type=jnp.float32)
    yt_ref[0, :, :HC] = jnp.bitwise_or(_key16(ya),
                                       lax.shift_left(_key16(yb), jnp.int32(16)))
    yt_ref[0, :, HC:] = jnp.zeros((N, HCP - HC), jnp.int32)


def _t1(xf, wt, b2):
    return pl.pallas_call(
        _t1_body,
        grid=(B,),
        in_specs=[
            pl.BlockSpec((1, C_IN, N), lambda i: (i, 0, 0)),
            pl.BlockSpec((C_OUT, 2 * C_IN), lambda i: (0, 0)),
            pl.BlockSpec((C_OUT, 1), lambda i: (0, 0)),
        ],
        out_specs=[
            pl.BlockSpec((1, N, K), lambda i: (i, 0, 0)),
            pl.BlockSpec((1, C_OUT, N), lambda i: (i, 0, 0)),
            pl.BlockSpec((1, N, HCP), lambda i: (i, 0, 0)),
        ],
        out_shape=[
            jax.ShapeDtypeStruct((B, N, K), jnp.int32),
            jax.ShapeDtypeStruct((B, C_OUT, N), jnp.bfloat16),
            jax.ShapeDtypeStruct((B, N, HCP), jnp.int32),
        ],
    )(xf, wt, b2)


def _sc_body(idx_hbm, yt_hbm, out_hbm,
             idx0, idx1, rows0, rows1, out0, out1, gsem, osem):
    wid = lax.axis_index("s") * 2 + lax.axis_index("c")
    base = NCH // NW
    rem = NCH % NW
    start = jnp.where(wid < rem, (base + 1) * wid, base * wid + rem)
    cnt = jnp.where(wid < rem, base + 1, base)

    def issue(c, idxv, rowsv):
        ch = start + c
        pltpu.sync_copy(idx_hbm.at[pl.ds(ch * IPC, IPC)], idxv)
        pltpu.async_copy(yt_hbm.at[idxv.at[pl.ds(0, IPH)]],
                         rowsv.at[pl.ds(0, IPH)], gsem)
        pltpu.async_copy(yt_hbm.at[idxv.at[pl.ds(IPH, IPH)]],
                         rowsv.at[pl.ds(IPH, IPH)], gsem)

    def wait_gather(rowsv):
        pltpu.make_async_copy(yt_hbm.at[pl.ds(0, IPC)], rowsv, gsem).wait()

    def compute(rowsv, outv):
        mask = jnp.int32(0xFFFF)
        sixteen = jnp.int32(16)

        def oblock(o, carry):
            sl = pl.ds(pl.multiple_of(o * 16, 16), 16)
            for r in range(CR):
                v = rowsv[r * K, sl]
                mlo = jnp.bitwise_and(v, mask)
                mhi = lax.shift_right_logical(v, sixteen)
                for k in range(1, K):
                    v = rowsv[r * K + k, sl]
                    mlo = jnp.maximum(mlo, jnp.bitwise_and(v, mask))
                    mhi = jnp.maximum(mhi, lax.shift_right_logical(v, sixteen))
                outv[r, sl] = jnp.bitwise_or(mlo, lax.shift_left(mhi, sixteen))
            return carry
        lax.fori_loop(0, HC // 16, oblock, 0)

    def out_copy(c, outv):
        ch = start + c
        pltpu.async_copy(outv, out_hbm.at[pl.ds(ch * CR, CR)], osem)

    def wait_out(outv):
        pltpu.make_async_copy(outv, out_hbm.at[pl.ds(0, CR)], osem).wait()

    issue(0, idx0, rows0)

    def pair(p, carry):
        c0 = 2 * p
        c1 = c0 + 1

        @pl.when(c1 < cnt)
        def _():
            issue(c1, idx1, rows1)

        wait_gather(rows0)

        @pl.when(p > 0)
        def _():
            wait_out(out0)

        compute(rows0, out0)
        out_copy(c0, out0)

        @pl.when(c1 < cnt)
        def _():
            @pl.when(c1 + 1 < cnt)
            def _():
                issue(c1 + 1, idx0, rows0)

            wait_gather(rows1)

            @pl.when(p > 0)
            def _():
                wait_out(out1)

            compute(rows1, out1)
            out_copy(c1, out1)

        return carry

    lax.fori_loop(0, (cnt + 1) // 2, pair, 0)
    wait_out(out0)
    wait_out(out1)


def _sc_gather_max(idx_flat, yt_flat):
    mesh = plsc.VectorSubcoreMesh(core_axis_name="c", subcore_axis_name="s",
                                  num_cores=2, num_subcores=16)
    fn = pl.kernel(
        _sc_body,
        out_type=jax.ShapeDtypeStruct((BN, HC), jnp.int32),
        mesh=mesh,
        scratch_types=[
            pltpu.VMEM((IPC,), jnp.int32),
            pltpu.VMEM((IPC,), jnp.int32),
            pltpu.VMEM((IPC, HCP), jnp.int32),
            pltpu.VMEM((IPC, HCP), jnp.int32),
            pltpu.VMEM((CR, HC), jnp.int32),
            pltpu.VMEM((CR, HC), jnp.int32),
            pltpu.SemaphoreType.DMA,
            pltpu.SemaphoreType.DMA,
        ],
    )
    return fn(idx_flat, yt_flat)


def _t2_body(a_ref, mt_ref, out_ref):
    mt = mt_ref[0]
    fa = _unkey16(jnp.bitwise_and(mt, jnp.int32(0xFFFF)))
    fb = _unkey16(lax.shift_right_logical(mt, jnp.int32(16)))
    a_lo = a_ref[0, :HC].astype(jnp.float32)
    a_hi = a_ref[0, HC:].astype(jnp.float32)
    lo = jnp.maximum(a_lo + jnp.transpose(fa), 0.0)
    hi = jnp.maximum(a_hi + jnp.transpose(fb), 0.0)
    out_ref[0, :HC] = lo.reshape(HC, H, W)
    out_ref[0, HC:] = hi.reshape(HC, H, W)


def _t2(a, mt):
    return pl.pallas_call(
        _t2_body,
        grid=(B,),
        in_specs=[
            pl.BlockSpec((1, C_OUT, N), lambda i: (i, 0, 0)),
            pl.BlockSpec((1, N, HC), lambda i: (i, 0, 0)),
        ],
        out_specs=pl.BlockSpec((1, C_OUT, H, W), lambda i: (i, 0, 0, 0)),
        out_shape=jax.ShapeDtypeStruct((B, C_OUT, H, W), jnp.float32),
    )(a, mt)


def kernel(x, Wt, b):
    xf = x.reshape(B, C_IN, N)
    b2 = b.reshape(C_OUT, 1)
    idx, a, ytp = _t1(xf, Wt, b2)
    idx_flat = idx.reshape(BN * K)
    ytp_flat = ytp.reshape(BN, HCP)
    mt = _sc_gather_max(idx_flat, ytp_flat)
    return _t2(a, mt.reshape(B, N, HC))


# speedup vs baseline: 1.2320x; 1.2320x over previous
import functools

import jax
import jax.numpy as jnp
from jax import lax
from jax.experimental import pallas as pl
from jax.experimental.pallas import tpu as pltpu
from jax.experimental.pallas import tpu_sc as plsc

B, C_IN, C_OUT, H, W, K = 4, 192, 384, 28, 28, 9
N = H * W
BN = B * N
KN = K - 1

NW = 32
HC = C_OUT // 2
HCP = 256
CR = 16
NCH = BN // CR
IPC = CR * K
IPH = IPC // 2


def _key16(v):
    bits = lax.bitcast_convert_type(v, jnp.int32)
    rnd = jnp.bitwise_and(lax.shift_right_logical(bits, jnp.int32(16)),
                          jnp.int32(1)) + jnp.int32(0x7FFF)
    u = lax.shift_right_logical(bits + rnd, jnp.int32(16))
    s = lax.shift_right_logical(u, jnp.int32(15))
    flip = jnp.bitwise_or(jnp.bitwise_and(jnp.int32(0) - s, jnp.int32(0xFFFF)),
                          jnp.int32(0x8000))
    return jnp.bitwise_xor(u, flip)


def _unkey16(k):
    u = jnp.where(k >= jnp.int32(0x8000),
                  jnp.bitwise_xor(k, jnp.int32(0x8000)),
                  jnp.bitwise_xor(k, jnp.int32(0xFFFF)))
    return lax.bitcast_convert_type(lax.shift_left(u, jnp.int32(16)),
                                    jnp.float32)


def _t1_body(x_ref, wt_ref, b_ref, idx_ref, a_ref, yt_ref):
    bidx = pl.program_id(0)
    x = x_ref[0]
    s = jnp.sum(x * x, axis=0, keepdims=True)
    xn = x / (jnp.sqrt(s) + 1e-12)

    g = lax.dot_general(xn, xn, (((0,), (0,)), ((), ())),
                        preferred_element_type=jnp.float32)
    sq = jnp.sum(xn * xn, axis=0, keepdims=True)
    dist = jnp.transpose(sq) + (-2.0 * g) + sq

    rows = lax.broadcasted_iota(jnp.int32, (N, N), 0)
    cols = lax.broadcasted_iota(jnp.int32, (N, N), 1)

    dq = lax.convert_element_type(
        jnp.minimum(jnp.maximum(dist, 0.0), 3.9) * 524288.0, jnp.int32)
    key_i = jnp.bitwise_or(lax.shift_left(dq, jnp.int32(10)), cols)
    bigf = lax.bitcast_convert_type(jnp.int32(0x7F000000), jnp.float32)
    key = jnp.where(rows == cols, bigf,
                    lax.bitcast_convert_type(key_i, jnp.float32))

    picked = []
    for _ in range(KN):
        m = jnp.min(key, axis=1, keepdims=True)
        am = jnp.bitwise_and(lax.bitcast_convert_type(m, jnp.int32),
                             jnp.int32(0x3FF))
        picked.append(am + bidx * N)
        key = jnp.where(key == m, bigf, key)
    picked.append(lax.broadcasted_iota(jnp.int32, (N, 1), 0) + bidx * N)
    idx_ref[0] = jnp.concatenate(picked, axis=1)

    w1 = wt_ref[:, :C_IN]
    w2 = wt_ref[:, C_IN:]
    a = lax.dot_general(w1 - w2, x, (((1,), (0,)), ((), ())),
                        preferred_element_type=jnp.float32)
    a_ref[0] = (a + b_ref[...]).astype(jnp.bfloat16)
    ya = lax.dot_general(x, w2[:HC], (((0,), (1,)), ((), ())),
                         preferred_element_type=jnp.float32)
    yb = lax.dot_general(x, w2[HC:], (((0,), (1,)), ((), ())),
                         preferred_element_type=jnp.float32)
    yt_ref[0, :, :HC] = jnp.bitwise_or(_key16(ya),
                                       lax.shift_left(_key16(yb), jnp.int32(16)))
    yt_ref[0, :, HC:] = jnp.zeros((N, HCP - HC), jnp.int32)


def _t1(xf, wt, b2, nb):
    return pl.pallas_call(
        _t1_body,
        grid=(nb,),
        in_specs=[
            pl.BlockSpec((1, C_IN, N), lambda i: (i, 0, 0)),
            pl.BlockSpec((C_OUT, 2 * C_IN), lambda i: (0, 0)),
            pl.BlockSpec((C_OUT, 1), lambda i: (0, 0)),
        ],
        out_specs=[
            pl.BlockSpec((1, N, K), lambda i: (i, 0, 0)),
            pl.BlockSpec((1, C_OUT, N), lambda i: (i, 0, 0)),
            pl.BlockSpec((1, N, HCP), lambda i: (i, 0, 0)),
        ],
        out_shape=[
            jax.ShapeDtypeStruct((nb, N, K), jnp.int32),
            jax.ShapeDtypeStruct((nb, C_OUT, N), jnp.bfloat16),
            jax.ShapeDtypeStruct((nb, N, HCP), jnp.int32),
        ],
    )(xf, wt, b2)


def _sc_body(nch, idx_hbm, yt_hbm, out_hbm,
             idx0, idx1, rows0, rows1, out0, out1, gsem, osem):
    wid = lax.axis_index("s") * 2 + lax.axis_index("c")
    base = nch // NW
    rem = nch % NW
    start = jnp.where(wid < rem, (base + 1) * wid, base * wid + rem)
    cnt = jnp.where(wid < rem, base + 1, base)

    def issue(c, idxv, rowsv):
        ch = start + c
        pltpu.sync_copy(idx_hbm.at[pl.ds(ch * IPC, IPC)], idxv)
        pltpu.async_copy(yt_hbm.at[idxv.at[pl.ds(0, IPH)]],
                         rowsv.at[pl.ds(0, IPH)], gsem)
        pltpu.async_copy(yt_hbm.at[idxv.at[pl.ds(IPH, IPH)]],
                         rowsv.at[pl.ds(IPH, IPH)], gsem)

    def wait_gather(rowsv):
        pltpu.make_async_copy(yt_hbm.at[pl.ds(0, IPC)], rowsv, gsem).wait()

    def compute(rowsv, outv):
        mask = jnp.int32(0xFFFF)
        sixteen = jnp.int32(16)

        def oblock(o, carry):
            sl = pl.ds(pl.multiple_of(o * 16, 16), 16)
            for r in range(CR):
                v = rowsv[r * K, sl]
                mlo = jnp.bitwise_and(v, mask)
                mhi = lax.shift_right_logical(v, sixteen)
                for k in range(1, K):
                    v = rowsv[r * K + k, sl]
                    mlo = jnp.maximum(mlo, jnp.bitwise_and(v, mask))
                    mhi = jnp.maximum(mhi, lax.shift_right_logical(v, sixteen))
                outv[r, sl] = jnp.bitwise_or(mlo, lax.shift_left(mhi, sixteen))
            return carry
        lax.fori_loop(0, HC // 16, oblock, 0)

    def out_copy(c, outv):
        ch = start + c
        pltpu.async_copy(outv, out_hbm.at[pl.ds(ch * CR, CR)], osem)

    def wait_out(outv):
        pltpu.make_async_copy(outv, out_hbm.at[pl.ds(0, CR)], osem).wait()

    issue(0, idx0, rows0)

    def pair(p, carry):
        c0 = 2 * p
        c1 = c0 + 1

        @pl.when(c1 < cnt)
        def _():
            issue(c1, idx1, rows1)

        wait_gather(rows0)

        @pl.when(p > 0)
        def _():
            wait_out(out0)

        compute(rows0, out0)
        out_copy(c0, out0)

        @pl.when(c1 < cnt)
        def _():
            @pl.when(c1 + 1 < cnt)
            def _():
                issue(c1 + 1, idx0, rows0)

            wait_gather(rows1)

            @pl.when(p > 0)
            def _():
                wait_out(out1)

            compute(rows1, out1)
            out_copy(c1, out1)

        return carry

    lax.fori_loop(0, (cnt + 1) // 2, pair, 0)
    wait_out(out0)
    wait_out(out1)


def _sc_gather_max(idx_flat, yt_flat):
    nrows = yt_flat.shape[0]
    mesh = plsc.VectorSubcoreMesh(core_axis_name="c", subcore_axis_name="s",
                                  num_cores=2, num_subcores=16)
    fn = pl.kernel(
        functools.partial(_sc_body, nrows // CR),
        out_type=jax.ShapeDtypeStruct((nrows, HC), jnp.int32),
        mesh=mesh,
        scratch_types=[
            pltpu.VMEM((IPC,), jnp.int32),
            pltpu.VMEM((IPC,), jnp.int32),
            pltpu.VMEM((IPC, HCP), jnp.int32),
            pltpu.VMEM((IPC, HCP), jnp.int32),
            pltpu.VMEM((CR, HC), jnp.int32),
            pltpu.VMEM((CR, HC), jnp.int32),
            pltpu.SemaphoreType.DMA,
            pltpu.SemaphoreType.DMA,
        ],
    )
    return fn(idx_flat, yt_flat)


def _t2_body(a_ref, mt_ref, out_ref):
    mt = mt_ref[0]
    fa = _unkey16(jnp.bitwise_and(mt, jnp.int32(0xFFFF)))
    fb = _unkey16(lax.shift_right_logical(mt, jnp.int32(16)))
    a_lo = a_ref[0, :HC].astype(jnp.float32)
    a_hi = a_ref[0, HC:].astype(jnp.float32)
    out_ref[0, :HC] = jnp.maximum(a_lo + jnp.transpose(fa), 0.0)
    out_ref[0, HC:] = jnp.maximum(a_hi + jnp.transpose(fb), 0.0)


def _t2(a, mt, nb):
    return pl.pallas_call(
        _t2_body,
        grid=(nb,),
        in_specs=[
            pl.BlockSpec((1, C_OUT, N), lambda i: (i, 0, 0)),
            pl.BlockSpec((1, N, HC), lambda i: (i, 0, 0)),
        ],
        out_specs=pl.BlockSpec((1, C_OUT, N), lambda i: (i, 0, 0)),
        out_shape=jax.ShapeDtypeStruct((nb, C_OUT, N), jnp.float32),
    )(a, mt)


def kernel(x, Wt, b):
    xf = x.reshape(B, C_IN, N)
    b2 = b.reshape(C_OUT, 1)
    nb = 2
    outs = []
    for h in range(B // nb):
        xh = lax.slice_in_dim(xf, h * nb, (h + 1) * nb, axis=0)
        idx, a, ytp = _t1(xh, Wt, b2, nb)
        mt = _sc_gather_max(idx.reshape(nb * N * K), ytp.reshape(nb * N, HCP))
        outs.append(_t2(a, mt.reshape(nb, N, HC), nb))
    out = jnp.concatenate(outs, axis=0)
    return out.reshape(B, C_OUT, H, W)
